# Initial kernel scaffold; baseline (speedup 1.0000x reference)
import jax
import jax.numpy as jnp
from jax.experimental import pallas as pl


def kernel(xyz, features, W1, b1, g1, be1, W2, b2, g2, be2):
    raise NotImplementedError("write your pallas kernel here")


# speedup vs baseline: 13.6346x; 13.6346x over previous
import functools

import jax
import jax.numpy as jnp
from jax import lax
from jax.experimental import pallas as pl
from jax.experimental.pallas import tpu as pltpu
from jax.experimental.pallas import tpu_sc as plsc

B = 8
N = 4096
S = 1024
K = 32
C_IN = 64
C_MID = 64
C_MIDP = 128
C_OUT = 128
EPS = 1e-3

SB = 128
MB = 256


def _fps_body(xx_ref, yy_ref, zz_ref, nxx_ref, nyy_ref, nzz_ref, dist_ref):
    iota = lax.broadcasted_iota(jnp.int32, (B, N), 1)
    iota_s = lax.broadcasted_iota(jnp.int32, (B, S), 1)
    dist_ref[...] = jnp.full((B, N), 1e10, jnp.float32)
    zero_s = jnp.zeros((B, S), jnp.float32)

    def body(i, carry):
        far, fx, fy, fz = carry
        xx = xx_ref[...]
        yy = yy_ref[...]
        zz = zz_ref[...]
        m = iota == far
        cx = jnp.sum(jnp.where(m, xx, 0.0), axis=1, keepdims=True)
        cy = jnp.sum(jnp.where(m, yy, 0.0), axis=1, keepdims=True)
        cz = jnp.sum(jnp.where(m, zz, 0.0), axis=1, keepdims=True)
        dx = xx - cx
        dy = yy - cy
        dz = zz - cz
        d2 = (dx * dx + dz * dz) + dy * dy
        dist = jnp.minimum(dist_ref[...], d2)
        dist_ref[...] = dist
        maxv = jnp.max(dist, axis=1, keepdims=True)
        nf = jnp.min(jnp.where(dist == maxv, iota, N), axis=1, keepdims=True)
        m2 = iota == nf
        nx = jnp.sum(jnp.where(m2, xx, 0.0), axis=1, keepdims=True)
        ny = jnp.sum(jnp.where(m2, yy, 0.0), axis=1, keepdims=True)
        nz = jnp.sum(jnp.where(m2, zz, 0.0), axis=1, keepdims=True)
        sel = iota_s == i
        fx = jnp.where(sel, nx, fx)
        fy = jnp.where(sel, ny, fy)
        fz = jnp.where(sel, nz, fz)
        return nf, fx, fy, fz

    _, fx, fy, fz = lax.fori_loop(
        0, S, body, (jnp.zeros((B, 1), jnp.int32), zero_s, zero_s, zero_s)
    )
    nxx_ref[...] = fx
    nyy_ref[...] = fy
    nzz_ref[...] = fz


def _fps(xx, yy, zz):
    out = [jax.ShapeDtypeStruct((B, S), jnp.float32)] * 3
    return pl.pallas_call(
        _fps_body,
        out_shape=out,
        scratch_shapes=[pltpu.VMEM((B, N), jnp.float32)],
    )(xx, yy, zz)


def _prep_body(cp_ref, w_ref, b_ref, g_ref):
    g_ref[...] = (
        jnp.dot(cp_ref[...], w_ref[...], preferred_element_type=jnp.float32)
        + b_ref[...]
    )


def _prep(concat_p, w1p, b1):
    return pl.pallas_call(
        _prep_body,
        grid=(B,),
        in_specs=[
            pl.BlockSpec((N, 128), lambda b: (b, 0)),
            pl.BlockSpec((128, C_MIDP), lambda b: (0, 0)),
            pl.BlockSpec((1, C_MIDP), lambda b: (0, 0)),
        ],
        out_specs=pl.BlockSpec((N, C_MIDP), lambda b: (b, 0)),
        out_shape=jax.ShapeDtypeStruct((B * N, C_MIDP), jnp.float32),
    )(concat_p, w1p, b1)


def _topk_body(xyzp_ref, nz_ref, w1x_ref, idx_ref, q_ref):
    b = pl.program_id(0)
    xyzp = xyzp_ref[0]
    nz = nz_ref[...]
    dot = jnp.dot(nz, xyzp, preferred_element_type=jnp.float32)
    xr = xyzp[0:1, :]
    yr = xyzp[1:2, :]
    zr = xyzp[2:3, :]
    xn = (xr * xr + zr * zr) + yr * yr
    c0 = nz[:, 0:1]
    c1 = nz[:, 1:2]
    c2 = nz[:, 2:3]
    sn = (c0 * c0 + c2 * c2) + c1 * c1
    d = jnp.maximum((sn - 2.0 * dot) + xn, 0.0)
    q_ref[...] = jnp.dot(nz, w1x_ref[...], preferred_element_type=jnp.float32)
    iota = lax.broadcasted_iota(jnp.int32, (SB, N), 1)
    base = b * N
    for k in range(K):
        minv = jnp.min(d, axis=1, keepdims=True)
        ci = jnp.min(jnp.where(d == minv, iota, N), axis=1, keepdims=True)
        idx_ref[:, k:k + 1] = ci + base
        d = jnp.where(iota == ci, jnp.float32(jnp.inf), d)


def _topk(xyzp, nz, w1x):
    nsb = S // SB
    return pl.pallas_call(
        _topk_body,
        grid=(B, nsb),
        in_specs=[
            pl.BlockSpec((1, 8, N), lambda b, s: (b, 0, 0)),
            pl.BlockSpec((SB, 8), lambda b, s: (b * nsb + s, 0)),
            pl.BlockSpec((8, C_MIDP), lambda b, s: (0, 0)),
        ],
        out_specs=[
            pl.BlockSpec((SB, K), lambda b, s: (b * nsb + s, 0)),
            pl.BlockSpec((SB, C_MIDP), lambda b, s: (b * nsb + s, 0)),
        ],
        out_shape=[
            jax.ShapeDtypeStruct((B * S, K), jnp.int32),
            jax.ShapeDtypeStruct((B * S, C_MIDP), jnp.float32),
        ],
    )(xyzp, nz, w1x)


_SC_CHUNK = 128


def _gather(table, idxg):
    info = plsc.get_sparse_core_info()
    nw = info.num_cores * info.num_subcores
    rows = idxg.shape[0]
    b_per_w = rows // nw
    nchunk = b_per_w // _SC_CHUNK
    mesh = plsc.VectorSubcoreMesh(core_axis_name="c", subcore_axis_name="s")

    @functools.partial(
        pl.kernel,
        mesh=mesh,
        out_type=jax.ShapeDtypeStruct((rows, C_MIDP), jnp.float32),
        scratch_types=[
            pltpu.VMEM((_SC_CHUNK,), jnp.int32),
            pltpu.VMEM((_SC_CHUNK, C_MIDP), jnp.float32),
            pltpu.SemaphoreType.DMA,
        ],
    )
    def k(table_hbm, idx_hbm, out_hbm, idx_v, rows_v, sem):
        wid = lax.axis_index("s") * info.num_cores + lax.axis_index("c")
        base = wid * b_per_w
        for c in range(nchunk):
            off = base + c * _SC_CHUNK
            pltpu.sync_copy(idx_hbm.at[pl.ds(off, _SC_CHUNK)], idx_v)
            pltpu.async_copy(table_hbm.at[idx_v], rows_v, sem).wait()
            pltpu.sync_copy(rows_v, out_hbm.at[pl.ds(off, _SC_CHUNK)])

    return k(table, idxg)


def _mlp_body(gg_ref, q_ref, s1_ref, be1_ref, w2_ref, b2_ref, s2_ref,
              be2_ref, o_ref):
    g = gg_ref[...]
    q = q_ref[...]
    qe = jnp.broadcast_to(q[:, None, :], (MB, K, C_MIDP)).reshape(MB * K, C_MIDP)
    h = jnp.maximum(g - qe, 0.0) * s1_ref[...] + be1_ref[...]
    z = jnp.dot(h, w2_ref[...], preferred_element_type=jnp.float32) + b2_ref[...]
    h2 = jnp.maximum(z, 0.0) * s2_ref[...] + be2_ref[...]
    o_ref[...] = jnp.max(h2.reshape(MB, K, C_OUT), axis=1)


def _mlp(gg, q, s1, be1, w2, b2, s2, be2):
    nmb = (B * S) // MB
    return pl.pallas_call(
        _mlp_body,
        grid=(nmb,),
        in_specs=[
            pl.BlockSpec((MB * K, C_MIDP), lambda i: (i, 0)),
            pl.BlockSpec((MB, C_MIDP), lambda i: (i, 0)),
            pl.BlockSpec((1, C_MIDP), lambda i: (0, 0)),
            pl.BlockSpec((1, C_MIDP), lambda i: (0, 0)),
            pl.BlockSpec((C_MIDP, C_OUT), lambda i: (0, 0)),
            pl.BlockSpec((1, C_OUT), lambda i: (0, 0)),
            pl.BlockSpec((1, C_OUT), lambda i: (0, 0)),
            pl.BlockSpec((1, C_OUT), lambda i: (0, 0)),
        ],
        out_specs=pl.BlockSpec((MB, C_OUT), lambda i: (i, 0)),
        out_shape=jax.ShapeDtypeStruct((B * S, C_OUT), jnp.float32),
    )(gg, q, s1, be1, w2, b2, s2, be2)


def kernel(xyz, features, W1, b1, g1, be1, W2, b2, g2, be2):
    xx = xyz[:, :, 0]
    yy = xyz[:, :, 1]
    zz = xyz[:, :, 2]

    nxx, nyy, nzz = _fps(xx, yy, zz)
    new_xyz = jnp.stack([nxx, nyy, nzz], axis=-1)

    zpad = jnp.zeros((B, 5, N), jnp.float32)
    xyzp = jnp.concatenate(
        [xx[:, None, :], yy[:, None, :], zz[:, None, :], zpad], axis=1
    )
    nz = jnp.concatenate(
        [new_xyz.reshape(B * S, 3), jnp.zeros((B * S, 5), jnp.float32)],
        axis=1,
    )
    cpad = C_MIDP - C_MID
    w1x = jnp.pad(W1[:3], ((0, 5), (0, cpad)))

    concat_p = jnp.concatenate(
        [xyz, features, jnp.zeros((B, N, 128 - 3 - C_IN), jnp.float32)],
        axis=-1,
    ).reshape(B * N, 128)
    w1p = jnp.pad(W1, ((0, 128 - 3 - C_IN), (0, cpad)))
    b1p = jnp.pad(b1, (0, cpad))[None, :]

    g_table = _prep(concat_p, w1p, b1p)
    idx, q = _topk(xyzp, nz, w1x)
    gg = _gather(g_table, idx.reshape(B * S * K))

    inv = 1.0 / jnp.sqrt(1.0 + EPS)
    s1 = jnp.pad(inv * g1, (0, cpad))[None, :]
    be1p = jnp.pad(be1, (0, cpad))[None, :]
    w2p = jnp.pad(W2, ((0, cpad), (0, 0)))
    s2 = (inv * g2)[None, :]
    out = _mlp(gg, q, s1, be1p, w2p, b2[None, :], s2, be2[None, :])
    return new_xyz, out.reshape(B, S, C_OUT)


# speedup vs baseline: 14.5029x; 1.0637x over previous
import functools

import jax
import jax.numpy as jnp
from jax import lax
from jax.experimental import pallas as pl
from jax.experimental.pallas import tpu as pltpu
from jax.experimental.pallas import tpu_sc as plsc

B = 8
N = 4096
S = 1024
K = 32
C_IN = 64
C_MID = 64
C_MIDP = 128
C_OUT = 128
EPS = 1e-3

SB = 128
MB = 256


def _fps_body(xx_ref, yy_ref, zz_ref, nxx_ref, nyy_ref, nzz_ref, dist_ref):
    iota = lax.broadcasted_iota(jnp.int32, (B, N), 1)
    iota_s = lax.broadcasted_iota(jnp.int32, (B, S), 1)
    dist_ref[...] = jnp.full((B, N), 1e10, jnp.float32)
    zero_s = jnp.zeros((B, S), jnp.float32)

    m0 = iota == 0
    xx0 = xx_ref[...]
    yy0 = yy_ref[...]
    zz0 = zz_ref[...]
    c0 = (
        jnp.sum(jnp.where(m0, xx0, 0.0), axis=1, keepdims=True),
        jnp.sum(jnp.where(m0, yy0, 0.0), axis=1, keepdims=True),
        jnp.sum(jnp.where(m0, zz0, 0.0), axis=1, keepdims=True),
    )

    def body(i, carry):
        cx, cy, cz, fx, fy, fz = carry
        xx = xx_ref[...]
        yy = yy_ref[...]
        zz = zz_ref[...]
        dx = xx - cx
        dy = yy - cy
        dz = zz - cz
        d2 = (dx * dx + dz * dz) + dy * dy
        dist = jnp.minimum(dist_ref[...], d2)
        dist_ref[...] = dist
        maxv = jnp.max(dist, axis=1, keepdims=True)
        nf = jnp.min(jnp.where(dist == maxv, iota, N), axis=1, keepdims=True)
        m2 = iota == nf
        nx = jnp.sum(jnp.where(m2, xx, 0.0), axis=1, keepdims=True)
        ny = jnp.sum(jnp.where(m2, yy, 0.0), axis=1, keepdims=True)
        nz = jnp.sum(jnp.where(m2, zz, 0.0), axis=1, keepdims=True)
        sel = iota_s == i
        fx = jnp.where(sel, nx, fx)
        fy = jnp.where(sel, ny, fy)
        fz = jnp.where(sel, nz, fz)
        return nx, ny, nz, fx, fy, fz

    _, _, _, fx, fy, fz = lax.fori_loop(
        0, S, body, (*c0, zero_s, zero_s, zero_s)
    )
    nxx_ref[...] = fx
    nyy_ref[...] = fy
    nzz_ref[...] = fz


def _fps(xx, yy, zz):
    out = [jax.ShapeDtypeStruct((B, S), jnp.float32)] * 3
    return pl.pallas_call(
        _fps_body,
        out_shape=out,
        scratch_shapes=[pltpu.VMEM((B, N), jnp.float32)],
    )(xx, yy, zz)


def _prep_body(cp_ref, w_ref, b_ref, g_ref):
    g_ref[...] = (
        jnp.dot(cp_ref[...], w_ref[...], preferred_element_type=jnp.float32)
        + b_ref[...]
    )


def _prep(concat_p, w1p, b1):
    return pl.pallas_call(
        _prep_body,
        grid=(B,),
        in_specs=[
            pl.BlockSpec((N, 128), lambda b: (b, 0)),
            pl.BlockSpec((128, C_MIDP), lambda b: (0, 0)),
            pl.BlockSpec((1, C_MIDP), lambda b: (0, 0)),
        ],
        out_specs=pl.BlockSpec((N, C_MIDP), lambda b: (b, 0)),
        out_shape=jax.ShapeDtypeStruct((B * N, C_MIDP), jnp.float32),
    )(concat_p, w1p, b1)


def _topk_body(xyzp_ref, nz_ref, w1x_ref, idx_ref, q_ref):
    b = pl.program_id(0)
    xyzp = xyzp_ref[0]
    nz = nz_ref[...]
    dot = jnp.dot(nz, xyzp, preferred_element_type=jnp.float32)
    xr = xyzp[0:1, :]
    yr = xyzp[1:2, :]
    zr = xyzp[2:3, :]
    xn = (xr * xr + zr * zr) + yr * yr
    c0 = nz[:, 0:1]
    c1 = nz[:, 1:2]
    c2 = nz[:, 2:3]
    sn = (c0 * c0 + c2 * c2) + c1 * c1
    d = jnp.maximum((sn - 2.0 * dot) + xn, 0.0)
    q_ref[...] = jnp.dot(nz, w1x_ref[...], preferred_element_type=jnp.float32)
    iota = lax.broadcasted_iota(jnp.int32, (SB, N), 1)
    base = b * N
    for k in range(K):
        minv = jnp.min(d, axis=1, keepdims=True)
        ci = jnp.min(jnp.where(d == minv, iota, N), axis=1, keepdims=True)
        idx_ref[:, k:k + 1] = ci + base
        d = jnp.where(iota == ci, jnp.float32(jnp.inf), d)


def _topk(xyzp, nz, w1x):
    nsb = S // SB
    return pl.pallas_call(
        _topk_body,
        grid=(B, nsb),
        in_specs=[
            pl.BlockSpec((1, 8, N), lambda b, s: (b, 0, 0)),
            pl.BlockSpec((SB, 8), lambda b, s: (b * nsb + s, 0)),
            pl.BlockSpec((8, C_MIDP), lambda b, s: (0, 0)),
        ],
        out_specs=[
            pl.BlockSpec((SB, K), lambda b, s: (b * nsb + s, 0)),
            pl.BlockSpec((SB, C_MIDP), lambda b, s: (b * nsb + s, 0)),
        ],
        out_shape=[
            jax.ShapeDtypeStruct((B * S, K), jnp.int32),
            jax.ShapeDtypeStruct((B * S, C_MIDP), jnp.float32),
        ],
    )(xyzp, nz, w1x)


_SC_CHUNK = 128


def _gather(table, idxg):
    info = plsc.get_sparse_core_info()
    nw = info.num_cores * info.num_subcores
    rows = idxg.shape[0]
    b_per_w = rows // nw
    nchunk = b_per_w // _SC_CHUNK
    mesh = plsc.VectorSubcoreMesh(core_axis_name="c", subcore_axis_name="s")

    @functools.partial(
        pl.kernel,
        mesh=mesh,
        out_type=jax.ShapeDtypeStruct((rows, C_MIDP), jnp.float32),
        scratch_types=[
            pltpu.VMEM((_SC_CHUNK,), jnp.int32),
            pltpu.VMEM((_SC_CHUNK, C_MIDP), jnp.float32),
            pltpu.SemaphoreType.DMA,
        ],
    )
    def k(table_hbm, idx_hbm, out_hbm, idx_v, rows_v, sem):
        wid = lax.axis_index("s") * info.num_cores + lax.axis_index("c")
        base = wid * b_per_w
        for c in range(nchunk):
            off = base + c * _SC_CHUNK
            pltpu.sync_copy(idx_hbm.at[pl.ds(off, _SC_CHUNK)], idx_v)
            pltpu.async_copy(table_hbm.at[idx_v], rows_v, sem).wait()
            pltpu.sync_copy(rows_v, out_hbm.at[pl.ds(off, _SC_CHUNK)])

    return k(table, idxg)


def _mlp_body(gg_ref, q_ref, s1_ref, be1_ref, w2_ref, b2_ref, s2_ref,
              be2_ref, o_ref):
    g = gg_ref[...]
    q = q_ref[...]
    qe = jnp.broadcast_to(q[:, None, :], (MB, K, C_MIDP)).reshape(MB * K, C_MIDP)
    h = jnp.maximum(g - qe, 0.0) * s1_ref[...] + be1_ref[...]
    z = jnp.dot(h, w2_ref[...], preferred_element_type=jnp.float32) + b2_ref[...]
    h2 = jnp.maximum(z, 0.0) * s2_ref[...] + be2_ref[...]
    o_ref[...] = jnp.max(h2.reshape(MB, K, C_OUT), axis=1)


def _mlp(gg, q, s1, be1, w2, b2, s2, be2):
    nmb = q.shape[0] // MB
    return pl.pallas_call(
        _mlp_body,
        grid=(nmb,),
        in_specs=[
            pl.BlockSpec((MB * K, C_MIDP), lambda i: (i, 0)),
            pl.BlockSpec((MB, C_MIDP), lambda i: (i, 0)),
            pl.BlockSpec((1, C_MIDP), lambda i: (0, 0)),
            pl.BlockSpec((1, C_MIDP), lambda i: (0, 0)),
            pl.BlockSpec((C_MIDP, C_OUT), lambda i: (0, 0)),
            pl.BlockSpec((1, C_OUT), lambda i: (0, 0)),
            pl.BlockSpec((1, C_OUT), lambda i: (0, 0)),
            pl.BlockSpec((1, C_OUT), lambda i: (0, 0)),
        ],
        out_specs=pl.BlockSpec((MB, C_OUT), lambda i: (i, 0)),
        out_shape=jax.ShapeDtypeStruct((q.shape[0], C_OUT), jnp.float32),
    )(gg, q, s1, be1, w2, b2, s2, be2)


def kernel(xyz, features, W1, b1, g1, be1, W2, b2, g2, be2):
    xx = xyz[:, :, 0]
    yy = xyz[:, :, 1]
    zz = xyz[:, :, 2]

    nxx, nyy, nzz = _fps(xx, yy, zz)
    new_xyz = jnp.stack([nxx, nyy, nzz], axis=-1)

    zpad = jnp.zeros((B, 5, N), jnp.float32)
    xyzp = jnp.concatenate(
        [xx[:, None, :], yy[:, None, :], zz[:, None, :], zpad], axis=1
    )
    nz = jnp.concatenate(
        [new_xyz.reshape(B * S, 3), jnp.zeros((B * S, 5), jnp.float32)],
        axis=1,
    )
    cpad = C_MIDP - C_MID
    w1x = jnp.pad(W1[:3], ((0, 5), (0, cpad)))

    concat_p = jnp.concatenate(
        [xyz, features, jnp.zeros((B, N, 128 - 3 - C_IN), jnp.float32)],
        axis=-1,
    ).reshape(B * N, 128)
    w1p = jnp.pad(W1, ((0, 128 - 3 - C_IN), (0, cpad)))
    b1p = jnp.pad(b1, (0, cpad))[None, :]

    g_table = _prep(concat_p, w1p, b1p)
    idx, q = _topk(xyzp, nz, w1x)

    inv = 1.0 / jnp.sqrt(1.0 + EPS)
    s1 = jnp.pad(inv * g1, (0, cpad))[None, :]
    be1p = jnp.pad(be1, (0, cpad))[None, :]
    w2p = jnp.pad(W2, ((0, cpad), (0, 0)))
    s2 = (inv * g2)[None, :]

    idx_flat = idx.reshape(B * S * K)
    half = (B * S) // 2
    outs = []
    for h in range(2):
        gg = _gather(g_table, idx_flat[h * half * K:(h + 1) * half * K])
        outs.append(
            _mlp(gg, q[h * half:(h + 1) * half], s1, be1p, w2p,
                 b2[None, :], s2, be2[None, :])
        )
    out = jnp.concatenate(outs, axis=0)
    return new_xyz, out.reshape(B, S, C_OUT)
